# baseline (speedup 1.0000x reference)
import functools

import jax
import jax.numpy as jnp
from jax.experimental import pallas as pl
from jax.experimental.pallas import tpu as pltpu

_EPS = 1e-5


def _in_scale_bias(s1, s2, n):
    mu = s1 * (1.0 / n)
    var = s2 * (1.0 / n) - mu * mu
    scale = jax.lax.rsqrt(var + _EPS)
    return scale, -mu * scale


def _shift_dn(v, Wo):
    z = jnp.zeros((1, v.shape[1]), v.dtype)
    s = jnp.concatenate([z, v[:-1]], axis=0)
    r = jax.lax.broadcasted_iota(jnp.int32, s.shape, 0)
    return jnp.where((r & (Wo - 1)) == 0, 0.0, s)


def _shift_up(v, Wo):
    z = jnp.zeros((1, v.shape[1]), v.dtype)
    s = jnp.concatenate([v[1:], z], axis=0)
    r = jax.lax.broadcasted_iota(jnp.int32, s.shape, 0)
    return jnp.where((r & (Wo - 1)) == (Wo - 1), 0.0, s)


def _store_bands(cols_ref, variants, shifts, M, S, C, bf16):
    z = jnp.zeros((S, C), bf16)
    for kh in range(3):
        for kw in range(3):
            b = (kh * 3 + kw) * C
            v = variants[kh][kw]
            if shifts[kh] == -1:
                cols_ref[0:S, b:b + C] = z
                cols_ref[S:M, b:b + C] = v[:M - S]
            elif shifts[kh] == 0:
                cols_ref[0:M, b:b + C] = v
            else:
                cols_ref[0:M - S, b:b + C] = v[S:]
                cols_ref[M - S:M, b:b + C] = z


def _block_kernel(x_ref, w1_ref, wsc_ref, w2_ref, out_ref, c1_ref, c2_ref,
                  *, H, W, Cin, Cout):
    f32, bf16 = jnp.float32, jnp.bfloat16
    Ho, Wo = H // 2, W // 2
    M = Ho * Wo

    x2 = x_ref[0].astype(f32)
    s = jnp.sum(x2, axis=(0, 1))
    ss = jnp.sum(x2 * x2, axis=(0, 1))
    scale, bias = _in_scale_bias(s[:Cin] + s[Cin:], ss[:Cin] + ss[Cin:], H * W)
    scale2 = jnp.concatenate([scale, scale])
    bias2 = jnp.concatenate([bias, bias])
    h = jnp.maximum(x2 * scale2 + bias2, 0.0)

    hv = h.reshape(Ho, 2, Wo, 2 * Cin)
    p0 = hv[:, 0].reshape(M, 2 * Cin)
    p1 = hv[:, 1].reshape(M, 2 * Cin)
    e0, o0 = p0[:, :Cin].astype(bf16), p0[:, Cin:].astype(bf16)
    e1, o1 = p1[:, :Cin].astype(bf16), p1[:, Cin:].astype(bf16)
    sh0 = _shift_dn(p0[:, Cin:], Wo).astype(bf16)
    sh1 = _shift_dn(p1[:, Cin:], Wo).astype(bf16)

    variants1 = [[sh1, e1, o1], [sh0, e0, o0], [sh1, e1, o1]]
    _store_bands(c1_ref, variants1, (-1, 0, 0), M, Wo, Cin, bf16)
    out1 = jnp.dot(c1_ref[...], w1_ref[...], preferred_element_type=f32)

    sc = jnp.dot(e0, wsc_ref[...], preferred_element_type=f32)

    t1 = jnp.sum(out1, axis=0)
    t2 = jnp.sum(out1 * out1, axis=0)
    scl2, b2 = _in_scale_bias(t1, t2, M)
    h2 = jnp.maximum(out1 * scl2 + b2, 0.0)

    c_m = _shift_dn(h2, Wo).astype(bf16)
    c_0 = h2.astype(bf16)
    c_p = _shift_up(h2, Wo).astype(bf16)
    variants2 = [[c_m, c_0, c_p]] * 3
    _store_bands(c2_ref, variants2, (-1, 0, 1), M, Wo, Cout, bf16)
    out = jnp.dot(c2_ref[...], w2_ref[...], preferred_element_type=f32) + sc
    out_ref[0] = jnp.transpose(out, (1, 0)).astype(out_ref.dtype)


def kernel(x, w1, w2, w_sc):
    scale = 2.0
    N, Cin, H, W = x.shape
    Cout = w1.shape[0]
    Ho, Wo = H // 2, W // 2
    bf16 = jnp.bfloat16

    w1_mat = (jnp.transpose(w1, (2, 3, 1, 0)).reshape(9 * Cin, Cout) * scale).astype(bf16)
    w2_mat = (jnp.transpose(w2, (2, 3, 1, 0)).reshape(9 * Cout, Cout) * scale).astype(bf16)
    wsc_mat = (jnp.transpose(w_sc[:, :, 0, 0], (1, 0)) * scale).astype(bf16)

    xh = jnp.transpose(x.astype(bf16), (0, 2, 3, 1)).reshape(N, H, W // 2, 2 * Cin)

    kfn = functools.partial(_block_kernel, H=H, W=W, Cin=Cin, Cout=Cout)
    out_t = pl.pallas_call(
        kfn,
        grid=(N,),
        in_specs=[
            pl.BlockSpec((1, H, W // 2, 2 * Cin), lambda n: (n, 0, 0, 0)),
            pl.BlockSpec((9 * Cin, Cout), lambda n: (0, 0)),
            pl.BlockSpec((Cin, Cout), lambda n: (0, 0)),
            pl.BlockSpec((9 * Cout, Cout), lambda n: (0, 0)),
        ],
        out_specs=pl.BlockSpec((1, Cout, Ho * Wo), lambda n: (n, 0, 0)),
        out_shape=jax.ShapeDtypeStruct((N, Cout, Ho * Wo), x.dtype),
        scratch_shapes=[
            pltpu.VMEM((Ho * Wo, 9 * Cin), bf16),
            pltpu.VMEM((Ho * Wo, 9 * Cout), bf16),
        ],
        compiler_params=pltpu.CompilerParams(
            dimension_semantics=("arbitrary",),
            vmem_limit_bytes=56 * 1024 * 1024,
        ),
    )(xh, w1_mat, wsc_mat, w2_mat)

    return out_t.reshape(N, Cout, Ho, Wo)


# speedup vs baseline: 1.1300x; 1.1300x over previous
import functools

import jax
import jax.numpy as jnp
from jax.experimental import pallas as pl
from jax.experimental.pallas import tpu as pltpu

_EPS = 1e-5


def _in_scale_bias(s1, s2, n):
    mu = s1 * (1.0 / n)
    var = s2 * (1.0 / n) - mu * mu
    scale = jax.lax.rsqrt(var + _EPS)
    return scale, -mu * scale


def _shift_dn(v, Wo):
    z = jnp.zeros((1, v.shape[1]), v.dtype)
    s = jnp.concatenate([z, v[:-1]], axis=0)
    r = jax.lax.broadcasted_iota(jnp.int32, s.shape, 0)
    return jnp.where((r & (Wo - 1)) == 0, 0.0, s)


def _shift_up(v, Wo):
    z = jnp.zeros((1, v.shape[1]), v.dtype)
    s = jnp.concatenate([v[1:], z], axis=0)
    r = jax.lax.broadcasted_iota(jnp.int32, s.shape, 0)
    return jnp.where((r & (Wo - 1)) == (Wo - 1), 0.0, s)


def _store_bands(cols_ref, variants, shifts, M, S, C, bf16):
    z = jnp.zeros((S, C), bf16)
    for kh in range(3):
        for kw in range(3):
            b = (kh * 3 + kw) * C
            v = variants[kh][kw]
            if shifts[kh] == -1:
                cols_ref[0:S, b:b + C] = z
                cols_ref[S:M, b:b + C] = v[:M - S]
            elif shifts[kh] == 0:
                cols_ref[0:M, b:b + C] = v
            else:
                cols_ref[0:M - S, b:b + C] = v[S:]
                cols_ref[M - S:M, b:b + C] = z


def _stage1_kernel(x_ref, w1_ref, wsc_ref, out1_ref, sc_ref, c1_ref,
                   *, H, W, Cin, Cout):
    f32, bf16 = jnp.float32, jnp.bfloat16
    Ho, Wo = H // 2, W // 2
    M = Ho * Wo

    x2 = x_ref[0]
    s = jnp.sum(x2, axis=(0, 1))
    ss = jnp.sum(x2 * x2, axis=(0, 1))
    scale, bias = _in_scale_bias(s[:Cin] + s[Cin:], ss[:Cin] + ss[Cin:], H * W)
    scale2 = jnp.concatenate([scale, scale])
    bias2 = jnp.concatenate([bias, bias])
    h = jnp.maximum(x2 * scale2 + bias2, 0.0)

    hv = h.reshape(Ho, 2, Wo, 2 * Cin)
    p0 = hv[:, 0].reshape(M, 2 * Cin)
    p1 = hv[:, 1].reshape(M, 2 * Cin)
    e0, o0 = p0[:, :Cin].astype(bf16), p0[:, Cin:].astype(bf16)
    e1, o1 = p1[:, :Cin].astype(bf16), p1[:, Cin:].astype(bf16)
    sh0 = _shift_dn(p0[:, Cin:], Wo).astype(bf16)
    sh1 = _shift_dn(p1[:, Cin:], Wo).astype(bf16)

    variants1 = [[sh1, e1, o1], [sh0, e0, o0], [sh1, e1, o1]]
    _store_bands(c1_ref, variants1, (-1, 0, 0), M, Wo, Cin, bf16)
    out1 = jnp.dot(c1_ref[...], w1_ref[...], preferred_element_type=f32)
    sc = jnp.dot(e0, wsc_ref[...], preferred_element_type=f32)

    out1_ref[0] = out1.astype(bf16)
    sc_ref[0] = sc.astype(bf16)


def _stage2_kernel(o1_ref, sc_ref, w2_ref, out_ref, c2_ref, *, Ho, Wo, Cout):
    f32, bf16 = jnp.float32, jnp.bfloat16
    M = Ho * Wo

    out1 = o1_ref[0].astype(f32)
    t1 = jnp.sum(out1, axis=0)
    t2 = jnp.sum(out1 * out1, axis=0)
    scl2, b2 = _in_scale_bias(t1, t2, M)
    h2 = jnp.maximum(out1 * scl2 + b2, 0.0)

    c_m = _shift_dn(h2, Wo).astype(bf16)
    c_0 = h2.astype(bf16)
    c_p = _shift_up(h2, Wo).astype(bf16)
    variants2 = [[c_m, c_0, c_p]] * 3
    _store_bands(c2_ref, variants2, (-1, 0, 1), M, Wo, Cout, bf16)

    out = jnp.dot(c2_ref[...], w2_ref[...], preferred_element_type=f32)
    out = out + sc_ref[0].astype(f32)
    out_ref[0] = jnp.transpose(out, (1, 0)).astype(out_ref.dtype)


def kernel(x, w1, w2, w_sc):
    scale = 2.0
    N, Cin, H, W = x.shape
    Cout = w1.shape[0]
    Ho, Wo = H // 2, W // 2
    bf16 = jnp.bfloat16

    w1_mat = (jnp.transpose(w1, (2, 3, 1, 0)).reshape(9 * Cin, Cout) * scale).astype(bf16)
    w2_mat = (jnp.transpose(w2, (2, 3, 1, 0)).reshape(9 * Cout, Cout) * scale).astype(bf16)
    wsc_mat = (jnp.transpose(w_sc[:, :, 0, 0], (1, 0)) * scale).astype(bf16)

    xh = jnp.transpose(x, (0, 2, 3, 1)).reshape(N, H, W // 2, 2 * Cin)

    cparams = pltpu.CompilerParams(
        dimension_semantics=("arbitrary",),
        vmem_limit_bytes=32 * 1024 * 1024,
    )

    k1 = functools.partial(_stage1_kernel, H=H, W=W, Cin=Cin, Cout=Cout)
    out1, sc = pl.pallas_call(
        k1,
        grid=(N,),
        in_specs=[
            pl.BlockSpec((1, H, W // 2, 2 * Cin), lambda n: (n, 0, 0, 0)),
            pl.BlockSpec((9 * Cin, Cout), lambda n: (0, 0)),
            pl.BlockSpec((Cin, Cout), lambda n: (0, 0)),
        ],
        out_specs=[
            pl.BlockSpec((1, Ho * Wo, Cout), lambda n: (n, 0, 0)),
            pl.BlockSpec((1, Ho * Wo, Cout), lambda n: (n, 0, 0)),
        ],
        out_shape=[
            jax.ShapeDtypeStruct((N, Ho * Wo, Cout), bf16),
            jax.ShapeDtypeStruct((N, Ho * Wo, Cout), bf16),
        ],
        scratch_shapes=[pltpu.VMEM((Ho * Wo, 9 * Cin), bf16)],
        compiler_params=cparams,
    )(xh, w1_mat, wsc_mat)

    k2 = functools.partial(_stage2_kernel, Ho=Ho, Wo=Wo, Cout=Cout)
    out_t = pl.pallas_call(
        k2,
        grid=(N,),
        in_specs=[
            pl.BlockSpec((1, Ho * Wo, Cout), lambda n: (n, 0, 0)),
            pl.BlockSpec((1, Ho * Wo, Cout), lambda n: (n, 0, 0)),
            pl.BlockSpec((9 * Cout, Cout), lambda n: (0, 0)),
        ],
        out_specs=pl.BlockSpec((1, Cout, Ho * Wo), lambda n: (n, 0, 0)),
        out_shape=jax.ShapeDtypeStruct((N, Cout, Ho * Wo), x.dtype),
        scratch_shapes=[pltpu.VMEM((Ho * Wo, 9 * Cout), bf16)],
        compiler_params=cparams,
    )(out1, sc, w2_mat)

    return out_t.reshape(N, Cout, Ho, Wo)


# speedup vs baseline: 2.3565x; 2.0853x over previous
import functools

import jax
import jax.numpy as jnp
from jax.experimental import pallas as pl
from jax.experimental.pallas import tpu as pltpu

_EPS = 1e-5
_WOFF = 8


def _in_relu(x, n):
    axes = tuple(range(x.ndim - 1))
    s = jnp.sum(x, axis=axes, keepdims=True)
    ss = jnp.sum(x * x, axis=axes, keepdims=True)
    mu = s * (1.0 / n)
    var = ss * (1.0 / n) - mu * mu
    scale = jax.lax.rsqrt(var + _EPS)
    return jnp.maximum(x * scale - mu * scale, 0.0)


def _im2col_patches(hp_ref, H, W, Cin, stride):
    Ho, Wo = H // stride, W // stride
    patches = []
    for kh in range(3):
        for kw in range(3):
            if stride == 1:
                p = hp_ref[kh:kh + Ho, _WOFF - 1 + kw:_WOFF - 1 + kw + Wo, :]
            else:
                p = hp_ref[pl.ds(kh, Ho, stride=stride),
                           pl.ds(_WOFF - 1 + kw, Wo, stride=stride), :]
            patches.append(p.reshape(Ho * Wo, Cin).astype(jnp.bfloat16))
    return patches


def _stage1_kernel(x_ref, w1_ref, wsc_ref, out1_ref, sc_ref, hp_ref,
                   *, H, W, Cin, Cout, stride):
    Ho, Wo = H // stride, W // stride

    hp_ref[...] = jnp.zeros_like(hp_ref)
    h = _in_relu(x_ref[0], H * W)
    hp_ref[1:H + 1, _WOFF:_WOFF + W, :] = h

    patches = _im2col_patches(hp_ref, H, W, Cin, stride)
    cols = jnp.concatenate(patches, axis=-1)
    out1 = jnp.dot(cols, w1_ref[...], preferred_element_type=jnp.float32)
    sc = jnp.dot(patches[4], wsc_ref[...], preferred_element_type=jnp.float32)

    out1_ref[0] = out1.reshape(Ho, Wo, Cout).astype(out1_ref.dtype)
    sc_ref[0] = sc.astype(sc_ref.dtype)


def _stage2_kernel(x_ref, sc_ref, w2_ref, out_ref, hp_ref, *, H, W, C):
    hp_ref[...] = jnp.zeros_like(hp_ref)
    h = _in_relu(x_ref[0].astype(jnp.float32), H * W)
    hp_ref[1:H + 1, _WOFF:_WOFF + W, :] = h

    patches = _im2col_patches(hp_ref, H, W, C, 1)
    cols = jnp.concatenate(patches, axis=-1)
    out = jnp.dot(cols, w2_ref[...], preferred_element_type=jnp.float32)
    out = out + sc_ref[0].astype(jnp.float32)
    out_ref[0] = out.astype(out_ref.dtype)


def kernel(x, w1, w2, w_sc):
    stride, scale = 2, 2.0
    N, Cin, H, W = x.shape
    Cout = w1.shape[0]
    Ho, Wo = H // stride, W // stride
    bf16 = jnp.bfloat16

    w1_mat = (jnp.transpose(w1, (2, 3, 1, 0)).reshape(9 * Cin, Cout) * scale).astype(bf16)
    w2_mat = (jnp.transpose(w2, (2, 3, 1, 0)).reshape(9 * Cout, Cout) * scale).astype(bf16)
    wsc_mat = (jnp.transpose(w_sc[:, :, 0, 0], (1, 0)) * scale).astype(bf16)

    x_nhwc = jnp.transpose(x, (0, 2, 3, 1))

    cparams = pltpu.CompilerParams(
        dimension_semantics=("parallel",),
        vmem_limit_bytes=32 * 1024 * 1024,
    )

    k1 = functools.partial(_stage1_kernel, H=H, W=W, Cin=Cin, Cout=Cout, stride=stride)
    out1, sc = pl.pallas_call(
        k1,
        grid=(N,),
        in_specs=[
            pl.BlockSpec((1, H, W, Cin), lambda n: (n, 0, 0, 0)),
            pl.BlockSpec((9 * Cin, Cout), lambda n: (0, 0)),
            pl.BlockSpec((Cin, Cout), lambda n: (0, 0)),
        ],
        out_specs=[
            pl.BlockSpec((1, Ho, Wo, Cout), lambda n: (n, 0, 0, 0)),
            pl.BlockSpec((1, Ho * Wo, Cout), lambda n: (n, 0, 0)),
        ],
        out_shape=[
            jax.ShapeDtypeStruct((N, Ho, Wo, Cout), bf16),
            jax.ShapeDtypeStruct((N, Ho * Wo, Cout), bf16),
        ],
        scratch_shapes=[pltpu.VMEM((H + 2, _WOFF + W + 8, Cin), jnp.float32)],
        compiler_params=cparams,
    )(x_nhwc, w1_mat, wsc_mat)

    k2 = functools.partial(_stage2_kernel, H=Ho, W=Wo, C=Cout)
    out = pl.pallas_call(
        k2,
        grid=(N,),
        in_specs=[
            pl.BlockSpec((1, Ho, Wo, Cout), lambda n: (n, 0, 0, 0)),
            pl.BlockSpec((1, Ho * Wo, Cout), lambda n: (n, 0, 0)),
            pl.BlockSpec((9 * Cout, Cout), lambda n: (0, 0)),
        ],
        out_specs=pl.BlockSpec((1, Ho * Wo, Cout), lambda n: (n, 0, 0)),
        out_shape=jax.ShapeDtypeStruct((N, Ho * Wo, Cout), x.dtype),
        scratch_shapes=[pltpu.VMEM((Ho + 2, _WOFF + Wo + 8, Cout), jnp.float32)],
        compiler_params=cparams,
    )(out1, sc, w2_mat)

    out = out.reshape(N, Ho, Wo, Cout)
    return jnp.transpose(out, (0, 3, 1, 2))
